# Initial kernel scaffold; baseline (speedup 1.0000x reference)
import jax
import jax.numpy as jnp
from jax.experimental import pallas as pl


def kernel(x, edge_index, W_rel0, b_rel0, W_root0, W_rel1, b_rel1, W_root1, W_rel2, b_rel2, W_root2):
    raise NotImplementedError("write your pallas kernel here")


# speedup vs baseline: 11.5282x; 11.5282x over previous
import functools

import jax
import jax.numpy as jnp
from jax import lax
from jax.experimental import pallas as pl
from jax.experimental.pallas import tpu as pltpu
from jax.experimental.pallas import tpu_sc as plsc

N = 10000
E = 320000
D = 128

NC = 2
NS = 16
NW = NC * NS

EPW = E // NW
K = 80
NCHUNK = EPW // K
NBUF = 3
ZR = 80
IBLK = 25
NIB = NCHUNK // IBLK

_mesh = plsc.VectorSubcoreMesh(core_axis_name="c", subcore_axis_name="s")


@functools.partial(
    pl.kernel,
    mesh=_mesh,
    out_type=[
        jax.ShapeDtypeStruct((N, D), jnp.float32),
        jax.ShapeDtypeStruct((N, D), jnp.float32),
    ],
    scratch_types=[
        pltpu.VMEM_SHARED((N, D), jnp.float32),
        pltpu.VMEM((IBLK, K), jnp.int32),
        pltpu.VMEM((IBLK, K), jnp.int32),
        pltpu.VMEM((IBLK, K), jnp.int32),
        pltpu.VMEM((IBLK, K), jnp.int32),
        pltpu.VMEM((K, D), jnp.float32),
        pltpu.VMEM((K, D), jnp.float32),
        pltpu.VMEM((K, D), jnp.float32),
        pltpu.SemaphoreType.DMA,
        pltpu.SemaphoreType.DMA,
        pltpu.SemaphoreType.DMA,
        pltpu.SemaphoreType.DMA,
        pltpu.SemaphoreType.DMA,
    ],
)
def _segsum(h_hbm, src_hbm, dst_hbm, p0_hbm, p1_hbm,
            acc, srci0, srci1, dsti0, dsti1, r0, r1, r2,
            s0, s1, s2, si0, si1):
    cid = lax.axis_index("c")
    sid = lax.axis_index("s")
    wid = cid * NS + sid
    rows = [r0, r1, r2]
    sems = [s0, s1, s2]
    srcis = [srci0, srci1]
    dstis = [dsti0, dsti1]
    semis = [si0, si1]
    zbuf = r2

    pltpu.sync_copy(src_hbm.at[wid, 0], srcis[0])
    pltpu.sync_copy(dst_hbm.at[wid, 0], dstis[0])
    pltpu.async_copy(src_hbm.at[wid, 1], srcis[1], semis[1])
    pltpu.async_copy(dst_hbm.at[wid, 1], dstis[1], semis[1])

    zeros = jnp.zeros((16,), jnp.float32)

    @pl.loop(0, ZR)
    def _(i):
        @pl.loop(0, D // 16)
        def _(j):
            zbuf[i, pl.ds(j * 16, 16)] = zeros

    @pl.loop(sid, N // ZR, step=NS)
    def _(r):
        pltpu.sync_copy(zbuf, acc.at[pl.ds(r * ZR, ZR)])

    plsc.subcore_barrier()

    def do_block(sI, dI):
        for i in range(NBUF - 1):
            pltpu.async_copy(h_hbm.at[sI.at[i]], rows[i], sems[i])

        @pl.loop(0, IBLK - 1, step=NBUF)
        def _(j):
            for b in range(NBUF):
                lc = j + b
                nf = lc + NBUF - 1
                pf = (NBUF - 1 + b) % NBUF

                @pl.when(nf < IBLK)
                def _():
                    pltpu.async_copy(h_hbm.at[sI.at[nf]], rows[pf], sems[pf])

                pltpu.make_async_copy(
                    h_hbm.at[sI.at[lc]], rows[b], sems[b]).wait()
                pltpu.sync_copy(rows[b], acc.at[dI.at[lc]], add=True)

        pltpu.make_async_copy(
            h_hbm.at[sI.at[IBLK - 1]], rows[0], sems[0]).wait()
        pltpu.sync_copy(rows[0], acc.at[dI.at[IBLK - 1]], add=True)

    @pl.loop(0, NIB + (NIB % 2), step=2)
    def _(ob):
        for p in range(2):
            ib = ob + p

            @pl.when(ib < NIB)
            def _():
                @pl.when(ib > 0)
                def _():
                    pltpu.make_async_copy(
                        src_hbm.at[wid, ib], srcis[p], semis[p]).wait()
                    pltpu.make_async_copy(
                        dst_hbm.at[wid, ib], dstis[p], semis[p]).wait()

                do_block(srcis[p], dstis[p])

                @pl.when(ib + 2 < NIB)
                def _():
                    pltpu.async_copy(
                        src_hbm.at[wid, ib + 2], srcis[p], semis[p])
                    pltpu.async_copy(
                        dst_hbm.at[wid, ib + 2], dstis[p], semis[p])

    plsc.subcore_barrier()

    @pl.when(cid == 0)
    def _():
        @pl.loop(sid, N // ZR, step=NS)
        def _(r):
            ds = pl.ds(r * ZR, ZR)
            pltpu.sync_copy(acc.at[ds], p0_hbm.at[ds])

    @pl.when(cid == 1)
    def _():
        @pl.loop(sid, N // ZR, step=NS)
        def _(r):
            ds = pl.ds(r * ZR, ZR)
            pltpu.sync_copy(acc.at[ds], p1_hbm.at[ds])


_BLK = 1000


def _combine_body(do_relu, p0_ref, p1_ref, h_ref, wrel_ref, b_ref, wroot_ref,
                  o_ref):
    s = p0_ref[...] + p1_ref[...]
    acc = lax.dot_general(
        s, wrel_ref[...], (((1,), (0,)), ((), ())),
        precision=lax.Precision.HIGHEST, preferred_element_type=jnp.float32)
    acc = acc + lax.dot_general(
        h_ref[...], wroot_ref[...], (((1,), (0,)), ((), ())),
        precision=lax.Precision.HIGHEST, preferred_element_type=jnp.float32)
    acc = acc + b_ref[...]
    if do_relu:
        acc = jnp.maximum(acc, 0.0)
    o_ref[...] = acc


def _combine(p0, p1, h, w_rel, b_rel, w_root, do_relu):
    return pl.pallas_call(
        functools.partial(_combine_body, do_relu),
        grid=(N // _BLK,),
        in_specs=[
            pl.BlockSpec((_BLK, D), lambda i: (i, 0)),
            pl.BlockSpec((_BLK, D), lambda i: (i, 0)),
            pl.BlockSpec((_BLK, D), lambda i: (i, 0)),
            pl.BlockSpec((D, D), lambda i: (0, 0)),
            pl.BlockSpec((1, D), lambda i: (0, 0)),
            pl.BlockSpec((D, D), lambda i: (0, 0)),
        ],
        out_specs=pl.BlockSpec((_BLK, D), lambda i: (i, 0)),
        out_shape=jax.ShapeDtypeStruct((N, D), jnp.float32),
    )(p0, p1, h, w_rel, b_rel.reshape(1, D), w_root)


def kernel(x, edge_index, W_rel0, b_rel0, W_root0, W_rel1, b_rel1, W_root1,
           W_rel2, b_rel2, W_root2):
    src = edge_index[0].reshape(NW, NIB, IBLK, K)
    dst = edge_index[1].reshape(NW, NIB, IBLK, K)
    h = x
    layers = [
        (W_rel0, b_rel0, W_root0, True),
        (W_rel1, b_rel1, W_root1, True),
        (W_rel2, b_rel2, W_root2, False),
    ]
    for w_rel, b_rel, w_root, do_relu in layers:
        p0, p1 = _segsum(h, src, dst)
        h = _combine(p0, p1, h, w_rel, b_rel, w_root, do_relu)
    return h


# speedup vs baseline: 11.6341x; 1.0092x over previous
import functools

import jax
import jax.numpy as jnp
from jax import lax
from jax.experimental import pallas as pl
from jax.experimental.pallas import tpu as pltpu
from jax.experimental.pallas import tpu_sc as plsc

N = 10000
E = 320000
D = 128

NC = 2
NS = 16
NW = NC * NS

EPW = E // NW
K = 80
NCHUNK = EPW // K
NBUF = 3
ZR = 80
IBLK = 25
NIB = NCHUNK // IBLK

_mesh = plsc.VectorSubcoreMesh(core_axis_name="c", subcore_axis_name="s")


@functools.partial(
    pl.kernel,
    mesh=_mesh,
    out_type=[
        jax.ShapeDtypeStruct((N, D), jnp.float32),
        jax.ShapeDtypeStruct((N, D), jnp.float32),
    ],
    scratch_types=[
        pltpu.VMEM_SHARED((N, D), jnp.float32),
        pltpu.VMEM((IBLK, K), jnp.int32),
        pltpu.VMEM((IBLK, K), jnp.int32),
        pltpu.VMEM((IBLK, K), jnp.int32),
        pltpu.VMEM((IBLK, K), jnp.int32),
        pltpu.VMEM((K, D), jnp.float32),
        pltpu.VMEM((K, D), jnp.float32),
        pltpu.VMEM((K, D), jnp.float32),
        pltpu.SemaphoreType.DMA,
        pltpu.SemaphoreType.DMA,
        pltpu.SemaphoreType.DMA,
        pltpu.SemaphoreType.DMA,
        pltpu.SemaphoreType.DMA,
        pltpu.SemaphoreType.DMA,
        pltpu.SemaphoreType.DMA,
        pltpu.SemaphoreType.DMA,
        pltpu.SemaphoreType.DMA,
    ],
)
def _segsum(h_hbm, src_hbm, dst_hbm, p0_hbm, p1_hbm,
            acc, srci0, srci1, dsti0, dsti1, r0, r1, r2,
            s0, s1, s2, c0, c1, c2, si0, si1, zsem):
    cid = lax.axis_index("c")
    sid = lax.axis_index("s")
    wid = cid * NS + sid
    rows = [r0, r1, r2]
    sems = [s0, s1, s2]
    ssems = [c0, c1, c2]
    srcis = [srci0, srci1]
    dstis = [dsti0, dsti1]
    semis = [si0, si1]
    zbuf = r2

    pltpu.sync_copy(src_hbm.at[wid, 0], srcis[0])
    pltpu.sync_copy(dst_hbm.at[wid, 0], dstis[0])
    pltpu.async_copy(src_hbm.at[wid, 1], srcis[1], semis[1])
    pltpu.async_copy(dst_hbm.at[wid, 1], dstis[1], semis[1])

    for i in range(NBUF - 1):
        pltpu.async_copy(h_hbm.at[srcis[0].at[i]], rows[i], sems[i])

    zeros = jnp.zeros((16,), jnp.float32)

    @pl.loop(0, ZR)
    def _(i):
        @pl.loop(0, D // 16)
        def _(j):
            zbuf[i, pl.ds(j * 16, 16)] = zeros

    @pl.loop(sid, N // ZR, step=NS)
    def _(r):
        pltpu.async_copy(zbuf, acc.at[pl.ds(r * ZR, ZR)], zsem)

    @pl.loop(sid, N // ZR, step=NS)
    def _(r):
        pltpu.make_async_copy(zbuf, acc.at[pl.ds(r * ZR, ZR)], zsem).wait()

    plsc.subcore_barrier()

    def do_block(sI, dI, prime):
        if prime:
            for i in range(NBUF - 1):
                pltpu.async_copy(h_hbm.at[sI.at[i]], rows[i], sems[i])

        @pl.loop(0, IBLK - 1, step=NBUF)
        def _(j):
            for b in range(NBUF):
                lc = j + b
                nf = lc + NBUF - 1
                pf = (NBUF - 1 + b) % NBUF

                @pl.when(nf < IBLK)
                def _():
                    def _wait_prev():
                        pltpu.make_async_copy(
                            rows[pf], acc.at[dI.at[lc - 1]],
                            ssems[pf]).wait()
                    if b == 0:
                        pl.when(lc > 0)(_wait_prev)
                    else:
                        _wait_prev()
                    pltpu.async_copy(h_hbm.at[sI.at[nf]], rows[pf], sems[pf])

                pltpu.make_async_copy(
                    h_hbm.at[sI.at[lc]], rows[b], sems[b]).wait()
                pltpu.async_copy(rows[b], acc.at[dI.at[lc]], ssems[b],
                                 add=True)

        pltpu.make_async_copy(
            h_hbm.at[sI.at[IBLK - 1]], rows[0], sems[0]).wait()
        pltpu.async_copy(rows[0], acc.at[dI.at[IBLK - 1]], ssems[0],
                         add=True)
        for c in range(IBLK - NBUF, IBLK):
            s = c % NBUF
            pltpu.make_async_copy(rows[s], acc.at[dI.at[c]], ssems[s]).wait()

    do_block(srcis[0], dstis[0], prime=False)
    pltpu.async_copy(src_hbm.at[wid, 2], srcis[0], semis[0])
    pltpu.async_copy(dst_hbm.at[wid, 2], dstis[0], semis[0])

    @pl.loop(1, NIB, step=2)
    def _(ob):
        for dp in range(2):
            ib = ob + dp
            p = (1 + dp) % 2

            @pl.when(ib < NIB)
            def _():
                pltpu.make_async_copy(
                    src_hbm.at[wid, ib], srcis[p], semis[p]).wait()
                pltpu.make_async_copy(
                    dst_hbm.at[wid, ib], dstis[p], semis[p]).wait()

                do_block(srcis[p], dstis[p], prime=True)

                @pl.when(ib + 2 < NIB)
                def _():
                    pltpu.async_copy(
                        src_hbm.at[wid, ib + 2], srcis[p], semis[p])
                    pltpu.async_copy(
                        dst_hbm.at[wid, ib + 2], dstis[p], semis[p])

    plsc.subcore_barrier()

    out_hbm = [p0_hbm, p1_hbm]
    for c in range(NC):
        @pl.when(cid == c)
        def _():
            @pl.loop(sid, N // ZR, step=NS)
            def _(r):
                ds = pl.ds(r * ZR, ZR)
                pltpu.async_copy(acc.at[ds], out_hbm[c].at[ds], zsem)

            @pl.loop(sid, N // ZR, step=NS)
            def _(r):
                ds = pl.ds(r * ZR, ZR)
                pltpu.make_async_copy(acc.at[ds], out_hbm[c].at[ds],
                                      zsem).wait()


_BLK = 1000


def _combine_body(do_relu, p0_ref, p1_ref, h_ref, wrel_ref, b_ref, wroot_ref,
                  o_ref):
    s = p0_ref[...] + p1_ref[...]
    acc = lax.dot_general(
        s, wrel_ref[...], (((1,), (0,)), ((), ())),
        precision=lax.Precision.HIGHEST, preferred_element_type=jnp.float32)
    acc = acc + lax.dot_general(
        h_ref[...], wroot_ref[...], (((1,), (0,)), ((), ())),
        precision=lax.Precision.HIGHEST, preferred_element_type=jnp.float32)
    acc = acc + b_ref[...]
    if do_relu:
        acc = jnp.maximum(acc, 0.0)
    o_ref[...] = acc


def _combine(p0, p1, h, w_rel, b_rel, w_root, do_relu):
    return pl.pallas_call(
        functools.partial(_combine_body, do_relu),
        grid=(N // _BLK,),
        in_specs=[
            pl.BlockSpec((_BLK, D), lambda i: (i, 0)),
            pl.BlockSpec((_BLK, D), lambda i: (i, 0)),
            pl.BlockSpec((_BLK, D), lambda i: (i, 0)),
            pl.BlockSpec((D, D), lambda i: (0, 0)),
            pl.BlockSpec((1, D), lambda i: (0, 0)),
            pl.BlockSpec((D, D), lambda i: (0, 0)),
        ],
        out_specs=pl.BlockSpec((_BLK, D), lambda i: (i, 0)),
        out_shape=jax.ShapeDtypeStruct((N, D), jnp.float32),
    )(p0, p1, h, w_rel, b_rel.reshape(1, D), w_root)


def kernel(x, edge_index, W_rel0, b_rel0, W_root0, W_rel1, b_rel1, W_root1,
           W_rel2, b_rel2, W_root2):
    src = edge_index[0].reshape(NW, NIB, IBLK, K)
    dst = edge_index[1].reshape(NW, NIB, IBLK, K)
    h = x
    layers = [
        (W_rel0, b_rel0, W_root0, True),
        (W_rel1, b_rel1, W_root1, True),
        (W_rel2, b_rel2, W_root2, False),
    ]
    for w_rel, b_rel, w_root, do_relu in layers:
        p0, p1 = _segsum(h, src, dst)
        h = _combine(p0, p1, h, w_rel, b_rel, w_root, do_relu)
    return h
